# initial kernel scaffold (unmeasured)
import jax
import jax.numpy as jnp
from jax import lax
from jax.experimental import pallas as pl
from jax.experimental.pallas import tpu as pltpu


def kernel(
    x,
):
    def body(*refs):
        pass

    out_shape = jax.ShapeDtypeStruct(..., jnp.float32)
    return pl.pallas_call(body, out_shape=out_shape)(...)


# baseline (device time: 29318 ns/iter reference)
import jax
import jax.numpy as jnp
from jax import lax
from jax.experimental import pallas as pl
from jax.experimental.pallas import tpu as pltpu

N_DEV = 32
LOG2_DEV = 5


def kernel(x):
    m_per, n = x.shape

    def body(x_ref, out_ref, acc_ref, recv_ref, send_sems, recv_sems):
        my_pos = lax.axis_index("i")

        xv = x_ref[:, :]
        val = jnp.max(xv, axis=0)
        rows = lax.broadcasted_iota(jnp.int32, (m_per, n), 0)
        masked = jnp.where(xv == val[None, :], rows, m_per)
        idx = jnp.min(masked, axis=0) + my_pos * m_per

        acc_ref[0, :] = val
        acc_ref[1, :] = idx.astype(jnp.float32)

        for k in range(LOG2_DEV):
            partner = my_pos ^ (1 << k)
            rdma = pltpu.make_async_remote_copy(
                src_ref=acc_ref,
                dst_ref=recv_ref.at[k],
                send_sem=send_sems.at[k],
                recv_sem=recv_sems.at[k],
                device_id=(partner,),
                device_id_type=pl.DeviceIdType.MESH,
            )
            rdma.start()
            rdma.wait()

            mv = acc_ref[0, :]
            mi = acc_ref[1, :]
            ov = recv_ref[k, 0, :]
            oi = recv_ref[k, 1, :]
            take = (ov > mv) | ((ov == mv) & (oi < mi))
            acc_ref[0, :] = jnp.where(take, ov, mv)
            acc_ref[1, :] = jnp.where(take, oi, mi)

        out_ref[:, :] = acc_ref[:, :]

    return pl.pallas_call(
        body,
        out_shape=jax.ShapeDtypeStruct((2, n), jnp.float32),
        in_specs=[pl.BlockSpec(memory_space=pltpu.VMEM)],
        out_specs=pl.BlockSpec(memory_space=pltpu.VMEM),
        scratch_shapes=[
            pltpu.VMEM((2, n), jnp.float32),
            pltpu.VMEM((LOG2_DEV, 2, n), jnp.float32),
            pltpu.SemaphoreType.DMA((LOG2_DEV,)),
            pltpu.SemaphoreType.DMA((LOG2_DEV,)),
        ],
    )(x)


# device time: 26223 ns/iter; 1.1180x vs baseline; 1.1180x over previous
import jax
import jax.numpy as jnp
from jax import lax
from jax.experimental import pallas as pl
from jax.experimental.pallas import tpu as pltpu

N_DEV = 32


def kernel(x):
    m_per, n = x.shape

    def body(x_ref, out_ref, recv_ref, send_sems, recv_sems):
        my_pos = lax.axis_index("i")

        xv = x_ref[:, :]
        val = jnp.max(xv, axis=0)
        rows = lax.broadcasted_iota(jnp.int32, (m_per, n), 0)
        masked = jnp.where(xv == val[None, :], rows, m_per)
        idx = jnp.min(masked, axis=0) + my_pos * m_per

        recv_ref[my_pos, 0, :] = val
        recv_ref[my_pos, 1, :] = idx.astype(jnp.float32)

        sends = []
        for p in range(1, N_DEV):
            peer = my_pos ^ p
            rdma = pltpu.make_async_remote_copy(
                src_ref=recv_ref.at[my_pos],
                dst_ref=recv_ref.at[my_pos],
                send_sem=send_sems.at[p],
                recv_sem=recv_sems.at[my_pos],
                device_id=(peer,),
                device_id_type=pl.DeviceIdType.MESH,
            )
            rdma.start()
            sends.append(rdma)

        for p in range(1, N_DEV):
            src = my_pos ^ p
            recv = pltpu.make_async_remote_copy(
                src_ref=recv_ref.at[src],
                dst_ref=recv_ref.at[src],
                send_sem=send_sems.at[p],
                recv_sem=recv_sems.at[src],
                device_id=(src,),
                device_id_type=pl.DeviceIdType.MESH,
            )
            recv.wait_recv()

        vals = recv_ref[:, 0, :]
        idxs = recv_ref[:, 1, :]
        m = jnp.max(vals, axis=0)
        gi = jnp.min(
            jnp.where(vals == m[None, :], idxs, float(N_DEV * m_per)), axis=0
        )
        out_ref[0, :] = m
        out_ref[1, :] = gi

        for rdma in sends:
            rdma.wait_send()

    return pl.pallas_call(
        body,
        out_shape=jax.ShapeDtypeStruct((2, n), jnp.float32),
        in_specs=[pl.BlockSpec(memory_space=pltpu.VMEM)],
        out_specs=pl.BlockSpec(memory_space=pltpu.VMEM),
        scratch_shapes=[
            pltpu.VMEM((N_DEV, 2, n), jnp.float32),
            pltpu.SemaphoreType.DMA((N_DEV,)),
            pltpu.SemaphoreType.DMA((N_DEV,)),
        ],
    )(x)


# device time: 23462 ns/iter; 1.2496x vs baseline; 1.1177x over previous
import jax
import jax.numpy as jnp
from jax import lax
from jax.experimental import pallas as pl
from jax.experimental.pallas import tpu as pltpu

N_DEV = 32
LOG2_DEV = 5


def kernel(x):
    m_per, n = x.shape

    def body(x_ref, out_ref, recv_ref, send_sems, recv_sems):
        my_pos = lax.axis_index("i")

        xv = x_ref[:, :]
        val = jnp.max(xv, axis=0)
        rows = lax.broadcasted_iota(jnp.int32, (m_per, n), 0)
        masked = jnp.where(xv == val[None, :], rows, m_per)
        idx = jnp.min(masked, axis=0) + my_pos * m_per

        recv_ref[my_pos, 0, :] = val
        recv_ref[my_pos, 1, :] = idx.astype(jnp.float32)

        barrier_sem = pltpu.get_barrier_semaphore()
        for r in range(LOG2_DEV):
            pl.semaphore_signal(
                barrier_sem,
                inc=1,
                device_id=((my_pos + (1 << r)) % N_DEV,),
                device_id_type=pl.DeviceIdType.MESH,
            )
            pl.semaphore_wait(barrier_sem, 1)

        sends = []
        for p in range(1, N_DEV):
            peer = my_pos ^ p
            rdma = pltpu.make_async_remote_copy(
                src_ref=recv_ref.at[my_pos],
                dst_ref=recv_ref.at[my_pos],
                send_sem=send_sems.at[p],
                recv_sem=recv_sems.at[my_pos],
                device_id=(peer,),
                device_id_type=pl.DeviceIdType.MESH,
            )
            rdma.start()
            sends.append(rdma)

        for p in range(1, N_DEV):
            src = my_pos ^ p
            recv = pltpu.make_async_remote_copy(
                src_ref=recv_ref.at[src],
                dst_ref=recv_ref.at[src],
                send_sem=send_sems.at[p],
                recv_sem=recv_sems.at[src],
                device_id=(src,),
                device_id_type=pl.DeviceIdType.MESH,
            )
            recv.wait_recv()

        vals = recv_ref[:, 0, :]
        idxs = recv_ref[:, 1, :]
        m = jnp.max(vals, axis=0)
        gi = jnp.min(
            jnp.where(vals == m[None, :], idxs, float(N_DEV * m_per)), axis=0
        )
        out_ref[0, :] = m
        out_ref[1, :] = gi

        for rdma in sends:
            rdma.wait_send()

    return pl.pallas_call(
        body,
        out_shape=jax.ShapeDtypeStruct((2, n), jnp.float32),
        in_specs=[pl.BlockSpec(memory_space=pltpu.VMEM)],
        out_specs=pl.BlockSpec(memory_space=pltpu.VMEM),
        scratch_shapes=[
            pltpu.VMEM((N_DEV, 2, n), jnp.float32),
            pltpu.SemaphoreType.DMA((N_DEV,)),
            pltpu.SemaphoreType.DMA((N_DEV,)),
        ],
        compiler_params=pltpu.CompilerParams(collective_id=0),
    )(x)


# device time: 21680 ns/iter; 1.3523x vs baseline; 1.0822x over previous
import jax
import jax.numpy as jnp
from jax import lax
from jax.experimental import pallas as pl
from jax.experimental.pallas import tpu as pltpu

N_DEV = 32
LOG2_DEV = 5


def kernel(x):
    m_per, n = x.shape

    def body(x_ref, out_ref, recv_ref, send_sems, recv_sems):
        my_pos = lax.axis_index("i")

        barrier_sem = pltpu.get_barrier_semaphore()
        rounds = [(1, 2, 3), (4, 8, 12), (16,)]

        n_chunks = 4
        nc = n // n_chunks
        rows = lax.broadcasted_iota(jnp.int32, (m_per, nc), 0)

        def compute_chunk(c):
            sl = pl.ds(c * nc, nc)
            xv = x_ref[:, sl]
            val = jnp.max(xv, axis=0)
            masked = jnp.where(xv == val[None, :], rows, m_per)
            idx = jnp.min(masked, axis=0) + my_pos * m_per
            recv_ref[my_pos, 0, sl] = val
            recv_ref[my_pos, 1, sl] = idx.astype(jnp.float32)

        for r, offs in enumerate(rounds):
            for o in offs:
                pl.semaphore_signal(
                    barrier_sem,
                    inc=1,
                    device_id=((my_pos + o) % N_DEV,),
                    device_id_type=pl.DeviceIdType.MESH,
                )
            compute_chunk(r)
            pl.semaphore_wait(barrier_sem, len(offs))
        compute_chunk(3)

        sends = []
        for p in range(1, N_DEV):
            peer = my_pos ^ p
            rdma = pltpu.make_async_remote_copy(
                src_ref=recv_ref.at[my_pos],
                dst_ref=recv_ref.at[my_pos],
                send_sem=send_sems.at[p],
                recv_sem=recv_sems.at[my_pos],
                device_id=(peer,),
                device_id_type=pl.DeviceIdType.MESH,
            )
            rdma.start()
            sends.append(rdma)

        for p in range(1, N_DEV):
            src = my_pos ^ p
            recv = pltpu.make_async_remote_copy(
                src_ref=recv_ref.at[src],
                dst_ref=recv_ref.at[src],
                send_sem=send_sems.at[p],
                recv_sem=recv_sems.at[src],
                device_id=(src,),
                device_id_type=pl.DeviceIdType.MESH,
            )
            recv.wait_recv()

        vals = recv_ref[:, 0, :]
        idxs = recv_ref[:, 1, :]
        m = jnp.max(vals, axis=0)
        gi = jnp.min(
            jnp.where(vals == m[None, :], idxs, float(N_DEV * m_per)), axis=0
        )
        out_ref[0, :] = m
        out_ref[1, :] = gi

        for rdma in sends:
            rdma.wait_send()

    return pl.pallas_call(
        body,
        out_shape=jax.ShapeDtypeStruct((2, n), jnp.float32),
        in_specs=[pl.BlockSpec(memory_space=pltpu.VMEM)],
        out_specs=pl.BlockSpec(memory_space=pltpu.VMEM),
        scratch_shapes=[
            pltpu.VMEM((N_DEV, 2, n), jnp.float32),
            pltpu.SemaphoreType.DMA((N_DEV,)),
            pltpu.SemaphoreType.DMA((N_DEV,)),
        ],
        compiler_params=pltpu.CompilerParams(collective_id=0),
    )(x)


# device time: 18448 ns/iter; 1.5892x vs baseline; 1.1752x over previous
import jax
import jax.numpy as jnp
from jax import lax
from jax.experimental import pallas as pl
from jax.experimental.pallas import tpu as pltpu

N_DEV = 32
LOG2_DEV = 5


def kernel(x):
    m_per, n = x.shape

    def body(x_ref, out_ref, recv_ref, send_sems, recv_sems):
        my_pos = lax.axis_index("i")

        barrier_sem = pltpu.get_barrier_semaphore()
        rounds = [(1, 2, 3), (4, 8, 12), (16,)]

        n_chunks = 4
        nc = n // n_chunks
        rows = lax.broadcasted_iota(jnp.int32, (m_per, nc), 0)

        def compute_chunk(c):
            sl = pl.ds(c * nc, nc)
            xv = x_ref[:, sl]
            val = jnp.max(xv, axis=0)
            masked = jnp.where(xv == val[None, :], rows, m_per)
            idx = jnp.min(masked, axis=0) + my_pos * m_per
            recv_ref[my_pos, 0, sl] = val
            recv_ref[my_pos, 1, sl] = idx.astype(jnp.float32)

        def signal_round(offs):
            for o in offs:
                pl.semaphore_signal(
                    barrier_sem,
                    inc=1,
                    device_id=((my_pos + o) % N_DEV,),
                    device_id_type=pl.DeviceIdType.MESH,
                )

        def send_to(offsets_behind):
            out = []
            for o in offsets_behind:
                peer = (my_pos - o) % N_DEV
                rdma = pltpu.make_async_remote_copy(
                    src_ref=recv_ref.at[my_pos],
                    dst_ref=recv_ref.at[my_pos],
                    send_sem=send_sems.at[o],
                    recv_sem=recv_sems.at[my_pos],
                    device_id=(peer,),
                    device_id_type=pl.DeviceIdType.MESH,
                )
                rdma.start()
                out.append(rdma)
            return out

        signal_round(rounds[0])
        compute_chunk(0)
        compute_chunk(1)
        pl.semaphore_wait(barrier_sem, 3)
        signal_round(rounds[1])
        compute_chunk(2)
        compute_chunk(3)
        pl.semaphore_wait(barrier_sem, 3)
        signal_round(rounds[2])
        sends = send_to(range(1, 16))
        pl.semaphore_wait(barrier_sem, 1)
        sends += send_to(range(16, N_DEV))

        for p in range(1, N_DEV):
            src = my_pos ^ p
            recv = pltpu.make_async_remote_copy(
                src_ref=recv_ref.at[src],
                dst_ref=recv_ref.at[src],
                send_sem=send_sems.at[p],
                recv_sem=recv_sems.at[src],
                device_id=(src,),
                device_id_type=pl.DeviceIdType.MESH,
            )
            recv.wait_recv()

        vals = recv_ref[:, 0, :]
        idxs = recv_ref[:, 1, :]
        m = jnp.max(vals, axis=0)
        gi = jnp.min(
            jnp.where(vals == m[None, :], idxs, float(N_DEV * m_per)), axis=0
        )
        out_ref[0, :] = m
        out_ref[1, :] = gi

        for rdma in sends:
            rdma.wait_send()

    return pl.pallas_call(
        body,
        out_shape=jax.ShapeDtypeStruct((2, n), jnp.float32),
        in_specs=[pl.BlockSpec(memory_space=pltpu.VMEM)],
        out_specs=pl.BlockSpec(memory_space=pltpu.VMEM),
        scratch_shapes=[
            pltpu.VMEM((N_DEV, 2, n), jnp.float32),
            pltpu.SemaphoreType.DMA((N_DEV,)),
            pltpu.SemaphoreType.DMA((N_DEV,)),
        ],
        compiler_params=pltpu.CompilerParams(collective_id=0),
    )(x)
